# baseline (speedup 1.0000x reference)
import functools
import math

import jax
import jax.numpy as jnp
from jax import lax
from jax.experimental import pallas as pl
from jax.experimental.pallas import tpu as pltpu
from jax.experimental.pallas import tpu_sc as plsc

D = 64
LANES = 16
SCALE = math.sqrt(D)
CHUNK = 400


@functools.lru_cache(maxsize=None)
def _build(b_total, vocab):
    info = plsc.get_sparse_core_info()
    nc, ns = info.num_cores, info.num_subcores
    nw = nc * ns
    b_per_w = b_total // nw
    n_chunks = b_per_w // CHUNK
    assert b_per_w % CHUNK == 0 and n_chunks % 2 == 0

    mesh = plsc.VectorSubcoreMesh(core_axis_name="c", subcore_axis_name="s")

    @functools.partial(
        pl.kernel,
        mesh=mesh,
        out_type=jax.ShapeDtypeStruct((b_total, D), jnp.float32),
        scratch_types=[
            pltpu.VMEM((n_chunks, CHUNK), jnp.int32),
            pltpu.VMEM((CHUNK, D), jnp.float32),
            pltpu.VMEM((CHUNK, D), jnp.float32),
            pltpu.VMEM((CHUNK, D), jnp.float32),
            pltpu.VMEM((CHUNK, D), jnp.float32),
            pltpu.SemaphoreType.DMA,
            pltpu.SemaphoreType.DMA,
            pltpu.SemaphoreType.DMA,
            pltpu.SemaphoreType.DMA,
        ],
        compiler_params=pltpu.CompilerParams(use_tc_tiling_on_sc=False),
    )
    def k(text_hbm, table_hbm, out_hbm, idx_v, gb0, gb1, ob0, ob1,
          gs0, gs1, os0, os1):
        wid = lax.axis_index("s") * nc + lax.axis_index("c")
        base = wid * b_per_w

        pltpu.sync_copy(text_hbm.at[pl.ds(wid * n_chunks, n_chunks)], idx_v)

        def gather_start(g, buf, sem):
            pltpu.async_copy(table_hbm.at[idx_v.at[g]], buf, sem)

        def gather_wait(buf, sem):
            pltpu.make_async_copy(table_hbm.at[idx_v.at[0]], buf, sem).wait()

        def out_start(g, buf, sem):
            pltpu.async_copy(buf, out_hbm.at[pl.ds(base + g * CHUNK, CHUNK)], sem)

        def out_wait(buf, sem):
            pltpu.make_async_copy(buf, out_hbm.at[pl.ds(base, CHUNK)], sem).wait()

        def scale(src, dst):
            def body(i, c):
                for j in range(D // LANES):
                    sl = pl.ds(j * LANES, LANES)
                    dst[i, sl] = src[i, sl] * SCALE
                return c

            lax.fori_loop(0, CHUNK, body, 0, unroll=8)

        def section(g, gb, gsem, ob, osem, gb_next, gsem_next, start_next,
                    wait_out):
            @pl.when(start_next)
            def _():
                gather_start(g + 1, gb_next, gsem_next)

            gather_wait(gb, gsem)

            @pl.when(wait_out)
            def _():
                out_wait(ob, osem)

            scale(gb, ob)
            out_start(g, ob, osem)

        gather_start(0, gb0, gs0)

        def loop_body(go, carry):
            g = 2 * go
            true_ = go >= 0
            section(g, gb0, gs0, ob0, os0, gb1, gs1, true_, go >= 1)
            section(g + 1, gb1, gs1, ob1, os1, gb0, gs0,
                    go < n_chunks // 2 - 1, go >= 1)
            return carry

        lax.fori_loop(0, n_chunks // 2, loop_body, 0)
        out_wait(ob0, os0)
        out_wait(ob1, os1)

    return k


def kernel(text, table):
    b_total = text.shape[0] * text.shape[1]
    text_flat = text.reshape(b_total).astype(jnp.int32)
    nw = 32
    b_per_w = b_total // nw
    n_chunks = b_per_w // CHUNK
    text2d = text_flat.reshape(nw * n_chunks, CHUNK)
    out = _build(b_total, table.shape[0])(text2d, table)
    return out.reshape(text.shape[0], text.shape[1], D)
